# baseline (speedup 1.0000x reference)
import functools

import jax
import jax.numpy as jnp
from jax import lax
from jax.experimental import pallas as pl
from jax.experimental.pallas import tpu as pltpu
from jax.experimental.pallas import tpu_sc as plsc

_B = 128
_U = 200
_D = 16
_N = _B * _U
_E = 409600
_NC = 2
_NS = 16
_TILE_EDGES = _E // (_NC * _NS)
_CHUNK = 128
_NCHUNK = _TILE_EDGES // _CHUNK
_ROWS_PER_TILE = _N // _NS
_ZROWS = 100


def _node_body(x4_ref, s2_ref, bu2_ref, w2_ref, c2_ref, tp_ref):
    h2 = jnp.maximum(
        jnp.dot(x4_ref[...], s2_ref[...], precision=lax.Precision.HIGHEST,
                preferred_element_type=jnp.float32) + bu2_ref[...], 0.0)
    tp_ref[...] = (
        jnp.dot(h2, w2_ref[...], precision=lax.Precision.HIGHEST,
                preferred_element_type=jnp.float32)
        + c2_ref[...])


def _node_phase(x4, s2, bu2, w2, c2):
    return pl.pallas_call(
        _node_body,
        out_shape=jax.ShapeDtypeStruct((_N // 2, 128), jnp.float32),
    )(x4, s2, bu2, w2, c2)


_NBUF = 4
_W = 4 * _D


def _sc_body(t_hbm, src_hbm, dst_hbm, ea_hbm, out_hbm,
             sall, dall, eall, rows, msg, zbuf, acc_sh, gsem):
    cid = lax.axis_index("c")
    sid = lax.axis_index("s")
    wid = cid * _NS + sid

    pltpu.sync_copy(src_hbm.at[pl.ds(wid * _NCHUNK, _NCHUNK)], sall)
    pltpu.sync_copy(dst_hbm.at[pl.ds(wid * _NCHUNK, _NCHUNK)], dall)
    pltpu.sync_copy(ea_hbm.at[pl.ds(wid * _NCHUNK, _NCHUNK)], eall)

    zero16 = jnp.zeros((_D,), jnp.float32)

    def zfill(j, carry):
        zbuf[j, pl.ds(0, _D)] = zero16
        zbuf[j, pl.ds(_D, _D)] = zero16
        return carry

    lax.fori_loop(0, _ZROWS, zfill, 0)
    row0 = sid * _ROWS_PER_TILE
    for k in range(_ROWS_PER_TILE // _ZROWS):
        pltpu.sync_copy(zbuf, acc_sh.at[pl.ds(row0 + k * _ZROWS, _ZROWS)])

    one16 = jnp.ones((_D,), jnp.float32)

    def ofill(j, carry):
        msg[j, pl.ds(_D, _D)] = one16
        return carry

    lax.fori_loop(0, _CHUNK, ofill, 0)
    plsc.subcore_barrier()

    def start_gather(ci, b):
        pltpu.async_copy(t_hbm.at[sall.at[ci]], rows.at[b], gsem)

    for p in range(_NBUF - 1):
        start_gather(p, p)

    def do_chunk(ci, b):
        nci = ci + _NBUF - 1

        @pl.when(nci < _NCHUNK)
        def _():
            start_gather(nci, (b + _NBUF - 1) % _NBUF)

        pltpu.make_async_copy(t_hbm.at[sall.at[ci]], rows.at[b], gsem).wait()

        def group_body(g, c2):
            base = g * _D
            ev = eall[ci, pl.ds(base, _D)]
            for k in range(_D):
                j = base + k
                p = rows[b, j, pl.ds(0, _D)]
                q = rows[b, j, pl.ds(_D, _D)]
                msg[j, pl.ds(0, _D)] = p * ev[k] + q
            return c2

        lax.fori_loop(0, _CHUNK // _D, group_body, 0)
        pltpu.sync_copy(msg, acc_sh.at[dall.at[ci]], add=True)

    def ring_body(h, carry):
        for b in range(_NBUF):
            do_chunk(h * _NBUF + b, b)
        return carry

    lax.fori_loop(0, _NCHUNK // _NBUF, ring_body, 0)
    plsc.subcore_barrier()

    pltpu.sync_copy(acc_sh.at[pl.ds(row0, _ROWS_PER_TILE)],
                    out_hbm.at[cid, pl.ds(row0, _ROWS_PER_TILE)])


def _edge_phase(t, src, dst, ea):
    mesh = plsc.VectorSubcoreMesh(core_axis_name="c", subcore_axis_name="s")
    f = pl.kernel(
        _sc_body,
        mesh=mesh,
        compiler_params=pltpu.CompilerParams(use_tc_tiling_on_sc=False),
        out_type=jax.ShapeDtypeStruct((_NC, _N, 2 * _D), jnp.float32),
        scratch_types=[
            pltpu.VMEM((_NCHUNK, _CHUNK), jnp.int32),
            pltpu.VMEM((_NCHUNK, _CHUNK), jnp.int32),
            pltpu.VMEM((_NCHUNK, _CHUNK), jnp.float32),
            pltpu.VMEM((_NBUF, _CHUNK, _W), jnp.float32),
            pltpu.VMEM((_CHUNK, 2 * _D), jnp.float32),
            pltpu.VMEM((_ZROWS, 2 * _D), jnp.float32),
            pltpu.VMEM_SHARED((_N, 2 * _D), jnp.float32),
            pltpu.SemaphoreType.DMA,
        ],
    )
    src2 = src.reshape(_E // _CHUNK, _CHUNK)
    dst2 = dst.reshape(_E // _CHUNK, _CHUNK)
    ea2 = ea.reshape(_E // _CHUNK, _CHUNK)
    return f(t, src2, dst2, ea2)


_CSTRIPE = _N // (_NC * _NS)


_LUTN = 4096


def _combine_body(acc_hbm, t_hbm, lut_hbm, out_hbm,
                  va, vb, vh, vo, vlut, sem):
    cid = lax.axis_index("c")
    sid = lax.axis_index("s")
    wid = cid * _NS + sid
    n0 = wid * _CSTRIPE
    pltpu.async_copy(acc_hbm.at[0, pl.ds(n0, _CSTRIPE)], va, sem)
    pltpu.async_copy(acc_hbm.at[1, pl.ds(n0, _CSTRIPE)], vb, sem)
    pltpu.async_copy(t_hbm.at[pl.ds(n0, _CSTRIPE)], vh, sem)
    pltpu.async_copy(lut_hbm, vlut, sem)
    pltpu.make_async_copy(acc_hbm.at[0, pl.ds(n0, _CSTRIPE)], va, sem).wait()
    pltpu.make_async_copy(acc_hbm.at[1, pl.ds(n0, _CSTRIPE)], vb, sem).wait()
    pltpu.make_async_copy(t_hbm.at[pl.ds(n0, _CSTRIPE)], vh, sem).wait()
    pltpu.make_async_copy(lut_hbm, vlut, sem).wait()

    def node_group(g, carry):
        for k in range(_D):
            j = g * _D + k
            s = va[j, pl.ds(0, _D)] + vb[j, pl.ds(0, _D)]
            c = va[j, pl.ds(_D, _D)] + vb[j, pl.ds(_D, _D)]
            idx = jnp.minimum(c, float(_LUTN - 1)).astype(jnp.int32)
            inv = plsc.load_gather(vlut, [idx])
            vo[j, :] = s * inv + vh[j, pl.ds(2 * _D, _D)]
        return carry

    lax.fori_loop(0, _CSTRIPE // _D, node_group, 0)
    pltpu.sync_copy(vo, out_hbm.at[pl.ds(n0, _CSTRIPE)])


def _combine(acc, t):
    mesh = plsc.VectorSubcoreMesh(core_axis_name="c", subcore_axis_name="s")
    f = pl.kernel(
        _combine_body,
        mesh=mesh,
        compiler_params=pltpu.CompilerParams(use_tc_tiling_on_sc=False,
                                             needs_layout_passes=False),
        out_type=jax.ShapeDtypeStruct((_N, _D), jnp.float32),
        scratch_types=[
            pltpu.VMEM((_CSTRIPE, 2 * _D), jnp.float32),
            pltpu.VMEM((_CSTRIPE, 2 * _D), jnp.float32),
            pltpu.VMEM((_CSTRIPE, _W), jnp.float32),
            pltpu.VMEM((_CSTRIPE, _D), jnp.float32),
            pltpu.VMEM((_LUTN,), jnp.float32),
            pltpu.SemaphoreType.DMA,
        ],
    )
    lut = jnp.concatenate(
        [jnp.zeros((1,), jnp.float32),
         1.0 / jnp.arange(1, _LUTN, dtype=jnp.float32)])
    return f(acc, t, lut)


def kernel(x, edge_index, edge_attribute, i, dummy,
           W_u, b_u, W_l1, b_l1, root, bias):
    x4 = x.reshape(_N // 2, 2)
    src = edge_index[0]
    dst = edge_index[1]
    ea = edge_attribute.reshape(_E)
    wall = jnp.concatenate(
        [W_l1.reshape(_D, _D), b_l1.reshape(_D, _D), root,
         jnp.zeros((_D, _D), jnp.float32)], axis=1)
    z = jnp.zeros((_D, _W), jnp.float32)
    w2 = jnp.concatenate(
        [jnp.concatenate([wall, z], axis=1),
         jnp.concatenate([z, wall], axis=1)], axis=0)
    wu = W_u.reshape(1, _D)
    zu = jnp.zeros((1, _D), jnp.float32)
    s2 = jnp.concatenate(
        [jnp.concatenate([wu, zu], axis=1),
         jnp.concatenate([zu, wu], axis=1)], axis=0)
    bu2 = jnp.tile(b_u.reshape(1, _D), (1, 2))
    cline = jnp.concatenate(
        [jnp.zeros((2 * _D,), jnp.float32), bias,
         jnp.zeros((_D,), jnp.float32)])
    c2 = jnp.tile(cline, 2).reshape(1, 128)
    tp = _node_phase(x4, s2, bu2, w2, c2)
    t = tp.reshape(_N, _W)
    acc = _edge_phase(t, src, dst, ea)
    return _combine(acc, t)


# speedup vs baseline: 1.4046x; 1.4046x over previous
import functools

import jax
import jax.numpy as jnp
from jax import lax
from jax.experimental import pallas as pl
from jax.experimental.pallas import tpu as pltpu
from jax.experimental.pallas import tpu_sc as plsc

_B = 128
_U = 200
_D = 16
_N = _B * _U
_E = 409600
_NC = 2
_NS = 16
_TILE_EDGES = _E // (_NC * _NS)
_CHUNK = 128
_NCHUNK = _TILE_EDGES // _CHUNK
_ROWS_PER_TILE = _N // _NS
_ZROWS = 100
_NBUF = 4


def _node_body(xf_ref, wu_ref, bu_ref, wpq_ref, root_ref, bias_ref,
               t_ref, hroot_ref):
    h = jnp.maximum(xf_ref[...] * wu_ref[...] + bu_ref[...], 0.0)
    t_ref[...] = jnp.dot(h, wpq_ref[...], preferred_element_type=jnp.float32)
    hroot_ref[...] = (
        jnp.dot(h, root_ref[...], preferred_element_type=jnp.float32)
        + bias_ref[...])


def _node_phase(xf, wu, bu, wpq, root, bias):
    return pl.pallas_call(
        _node_body,
        out_shape=(
            jax.ShapeDtypeStruct((_N, 2 * _D), jnp.float32),
            jax.ShapeDtypeStruct((_N, _D), jnp.float32),
        ),
    )(xf, wu, bu, wpq, root, bias)


def _sc_body(t_hbm, src_hbm, dst_hbm, ea_hbm, out_hbm,
             sall, dall, eall, rows, msg, zbuf, acc_sh, gsem):
    cid = lax.axis_index("c")
    sid = lax.axis_index("s")
    wid = cid * _NS + sid

    pltpu.sync_copy(src_hbm.at[pl.ds(wid * _NCHUNK, _NCHUNK)], sall)
    pltpu.sync_copy(dst_hbm.at[pl.ds(wid * _NCHUNK, _NCHUNK)], dall)
    pltpu.sync_copy(ea_hbm.at[pl.ds(wid * _NCHUNK, _NCHUNK)], eall)

    zero16 = jnp.zeros((_D,), jnp.float32)

    def zfill(j, carry):
        zbuf[j, pl.ds(0, _D)] = zero16
        zbuf[j, pl.ds(_D, _D)] = zero16
        return carry

    lax.fori_loop(0, _ZROWS, zfill, 0)
    row0 = sid * _ROWS_PER_TILE
    for k in range(_ROWS_PER_TILE // _ZROWS):
        pltpu.sync_copy(zbuf, acc_sh.at[pl.ds(row0 + k * _ZROWS, _ZROWS)])

    one16 = jnp.ones((_D,), jnp.float32)

    def ofill(j, carry):
        msg[j, pl.ds(_D, _D)] = one16
        return carry

    lax.fori_loop(0, _CHUNK, ofill, 0)
    plsc.subcore_barrier()

    def start_gather(ci, b):
        pltpu.async_copy(t_hbm.at[sall.at[ci]], rows.at[b], gsem)

    for p in range(_NBUF - 1):
        start_gather(p, p)

    def do_chunk(ci, b):
        nci = ci + _NBUF - 1

        @pl.when(nci < _NCHUNK)
        def _():
            start_gather(nci, (b + _NBUF - 1) % _NBUF)

        pltpu.make_async_copy(t_hbm.at[sall.at[ci]], rows.at[b], gsem).wait()

        def group_body(g, c2):
            base = g * _D
            ev = eall[ci, pl.ds(base, _D)]
            for k in range(_D):
                j = base + k
                p = rows[b, j, pl.ds(0, _D)]
                q = rows[b, j, pl.ds(_D, _D)]
                msg[j, pl.ds(0, _D)] = p * ev[k] + q
            return c2

        lax.fori_loop(0, _CHUNK // _D, group_body, 0)
        pltpu.sync_copy(msg, acc_sh.at[dall.at[ci]], add=True)

    def ring_body(h, carry):
        for b in range(_NBUF):
            do_chunk(h * _NBUF + b, b)
        return carry

    lax.fori_loop(0, _NCHUNK // _NBUF, ring_body, 0)
    plsc.subcore_barrier()

    pltpu.sync_copy(acc_sh.at[pl.ds(row0, _ROWS_PER_TILE)],
                    out_hbm.at[cid, pl.ds(row0, _ROWS_PER_TILE)])


def _edge_phase(t, src, dst, ea):
    mesh = plsc.VectorSubcoreMesh(core_axis_name="c", subcore_axis_name="s")
    f = pl.kernel(
        _sc_body,
        mesh=mesh,
        compiler_params=pltpu.CompilerParams(use_tc_tiling_on_sc=False),
        out_type=jax.ShapeDtypeStruct((_NC, _N, 2 * _D), jnp.float32),
        scratch_types=[
            pltpu.VMEM((_NCHUNK, _CHUNK), jnp.int32),
            pltpu.VMEM((_NCHUNK, _CHUNK), jnp.int32),
            pltpu.VMEM((_NCHUNK, _CHUNK), jnp.float32),
            pltpu.VMEM((_NBUF, _CHUNK, 2 * _D), jnp.float32),
            pltpu.VMEM((_CHUNK, 2 * _D), jnp.float32),
            pltpu.VMEM((_ZROWS, 2 * _D), jnp.float32),
            pltpu.VMEM_SHARED((_N, 2 * _D), jnp.float32),
            pltpu.SemaphoreType.DMA,
        ],
    )
    src2 = src.reshape(_E // _CHUNK, _CHUNK)
    dst2 = dst.reshape(_E // _CHUNK, _CHUNK)
    ea2 = ea.reshape(_E // _CHUNK, _CHUNK)
    return f(t, src2, dst2, ea2)


_CSTRIPE = _N // (_NC * _NS)
_LUTN = 4096


def _combine_body(acc_hbm, hroot_hbm, lut_hbm, out_hbm,
                  va, vb, vh, vo, vlut, sem):
    cid = lax.axis_index("c")
    sid = lax.axis_index("s")
    wid = cid * _NS + sid
    n0 = wid * _CSTRIPE
    pltpu.async_copy(acc_hbm.at[0, pl.ds(n0, _CSTRIPE)], va, sem)
    pltpu.async_copy(acc_hbm.at[1, pl.ds(n0, _CSTRIPE)], vb, sem)
    pltpu.async_copy(hroot_hbm.at[pl.ds(n0, _CSTRIPE)], vh, sem)
    pltpu.async_copy(lut_hbm, vlut, sem)
    pltpu.make_async_copy(acc_hbm.at[0, pl.ds(n0, _CSTRIPE)], va, sem).wait()
    pltpu.make_async_copy(acc_hbm.at[1, pl.ds(n0, _CSTRIPE)], vb, sem).wait()
    pltpu.make_async_copy(hroot_hbm.at[pl.ds(n0, _CSTRIPE)], vh, sem).wait()
    pltpu.make_async_copy(lut_hbm, vlut, sem).wait()

    def node_group(g, carry):
        for k in range(_D):
            j = g * _D + k
            s = va[j, pl.ds(0, _D)] + vb[j, pl.ds(0, _D)]
            c = va[j, pl.ds(_D, _D)] + vb[j, pl.ds(_D, _D)]
            idx = jnp.minimum(c, float(_LUTN - 1)).astype(jnp.int32)
            inv = plsc.load_gather(vlut, [idx])
            vo[j, :] = s * inv + vh[j, :]
        return carry

    lax.fori_loop(0, _CSTRIPE // _D, node_group, 0)
    pltpu.sync_copy(vo, out_hbm.at[pl.ds(n0, _CSTRIPE)])


def _combine(acc, hroot):
    mesh = plsc.VectorSubcoreMesh(core_axis_name="c", subcore_axis_name="s")
    f = pl.kernel(
        _combine_body,
        mesh=mesh,
        compiler_params=pltpu.CompilerParams(use_tc_tiling_on_sc=False,
                                             needs_layout_passes=False),
        out_type=jax.ShapeDtypeStruct((_N, _D), jnp.float32),
        scratch_types=[
            pltpu.VMEM((_CSTRIPE, 2 * _D), jnp.float32),
            pltpu.VMEM((_CSTRIPE, 2 * _D), jnp.float32),
            pltpu.VMEM((_CSTRIPE, _D), jnp.float32),
            pltpu.VMEM((_CSTRIPE, _D), jnp.float32),
            pltpu.VMEM((_LUTN,), jnp.float32),
            pltpu.SemaphoreType.DMA,
        ],
    )
    lut = jnp.concatenate(
        [jnp.zeros((1,), jnp.float32),
         1.0 / jnp.arange(1, _LUTN, dtype=jnp.float32)])
    return f(acc, hroot, lut)


def kernel(x, edge_index, edge_attribute, i, dummy,
           W_u, b_u, W_l1, b_l1, root, bias):
    xf = x.reshape(_N, 1)
    src = edge_index[0]
    dst = edge_index[1]
    ea = edge_attribute.reshape(_E)
    wpq = jnp.concatenate(
        [W_l1.reshape(_D, _D), b_l1.reshape(_D, _D)], axis=1)
    t, hroot = _node_phase(xf, W_u, b_u.reshape(1, _D), wpq,
                           root, bias.reshape(1, _D))
    acc = _edge_phase(t, src, dst, ea)
    return _combine(acc, hroot)
